# Initial kernel scaffold; baseline (speedup 1.0000x reference)
import jax
import jax.numpy as jnp
from jax.experimental import pallas as pl


def kernel(x, emb_table, temporal_table):
    raise NotImplementedError("write your pallas kernel here")


# speedup vs baseline: 20.9335x; 20.9335x over previous
import functools

import jax
import jax.numpy as jnp
from jax import lax
from jax.experimental import pallas as pl
from jax.experimental.pallas import tpu as pltpu
from jax.experimental.pallas import tpu_sc as plsc

_VOCAB = 100000
_SENT = 20
_MEM = 50
_EMB = 32
_BATCH = 1024

_NW = 32
_NSEG = _BATCH * _MEM
_SEG_W = _NSEG // _NW
_CHUNK = 32
_ROWS_CHUNK = _CHUNK * _SENT
_IDXROW = 128
_STREAMS = _ROWS_CHUNK // _IDXROW
_NCHUNK = _SEG_W // _CHUNK
_IDX_ROWS_W = _SEG_W * _SENT // _IDXROW

_SCALE = 4.0 / (_EMB * _SENT)
_W = [float((s + 1) - (_SENT + 1) / 2.0) for s in range(_SENT - 1)]


def _sc_body(idx_hbm, emb_hbm, temp_hbm, out_hbm,
             idx_v, rows_v, out_v, temp_v, sem0, sem1):
    wid = lax.axis_index("s") * 2 + lax.axis_index("c")

    pltpu.sync_copy(idx_hbm.at[wid], idx_v)
    pltpu.sync_copy(temp_hbm, temp_v)

    sems = (sem0, sem1)

    lane = lax.iota(jnp.int32, 16).astype(jnp.float32)
    cvec = [(lane + float(h * 16) - (_EMB - 1) / 2.0) * _SCALE
            for h in range(2)]

    def fire(c, b):
        for j in range(_STREAMS):
            pltpu.make_async_copy(
                emb_hbm.at[idx_v.at[c * _STREAMS + j]],
                rows_v.at[b, pl.ds(j * _IDXROW, _IDXROW)],
                sems[b],
            ).start()

    def drain(b):
        for j in range(_STREAMS):
            pltpu.make_async_copy(
                emb_hbm.at[idx_v.at[j]],
                rows_v.at[b, pl.ds(j * _IDXROW, _IDXROW)],
                sems[b],
            ).wait()

    def compute(c, b):
        def seg_body(g, carry):
            seg = c * _CHUNK + g
            m = lax.rem(seg, _MEM)
            rbase = g * _SENT
            for h in range(2):
                sl = pl.ds(h * 16, 16)
                acc = rows_v[b, rbase, sl] * _W[0]
                for s in range(1, _SENT - 1):
                    acc = acc + rows_v[b, rbase + s, sl] * _W[s]
                out_v[seg, sl] = (acc * cvec[h]
                                  + rows_v[b, rbase + _SENT - 1, sl]
                                  + temp_v[m, sl])
            return carry
        lax.fori_loop(0, _CHUNK, seg_body, 0)

    fire(0, 0)
    fire(1, 1)

    def loop_body(i, carry):
        for b in range(2):
            c = 2 * i + b
            drain(b)
            compute(c, b)
            fire(c + 2, b)
        return carry

    lax.fori_loop(0, (_NCHUNK - 2) // 2, loop_body, 0)

    for b in range(2):
        drain(b)
        compute(_NCHUNK - 2 + b, b)

    pltpu.sync_copy(out_v, out_hbm.at[pl.ds(wid * _SEG_W, _SEG_W)])


_sc_call = pl.kernel(
    _sc_body,
    out_type=jax.ShapeDtypeStruct((_NSEG, _EMB), jnp.float32),
    mesh=plsc.VectorSubcoreMesh(core_axis_name="c", subcore_axis_name="s"),
    scratch_types=[
        pltpu.VMEM((_IDX_ROWS_W, _IDXROW), jnp.int32),
        pltpu.VMEM((2, _ROWS_CHUNK, _EMB), jnp.float32),
        pltpu.VMEM((_SEG_W, _EMB), jnp.float32),
        pltpu.VMEM((_MEM, _EMB), jnp.float32),
        pltpu.SemaphoreType.DMA,
        pltpu.SemaphoreType.DMA,
    ],
    compiler_params=pltpu.CompilerParams(use_tc_tiling_on_sc=False),
)


@jax.jit
def kernel(x, emb_table, temporal_table):
    idx = x.astype(jnp.int32).reshape(_NW, _IDX_ROWS_W, _IDXROW)
    out = _sc_call(idx, emb_table, temporal_table)
    return out.reshape(_BATCH, _MEM, _EMB)


# speedup vs baseline: 20.9521x; 1.0009x over previous
import jax
import jax.numpy as jnp
from jax import lax
from jax.experimental import pallas as pl
from jax.experimental.pallas import tpu as pltpu
from jax.experimental.pallas import tpu_sc as plsc

_VOCAB = 100000
_SENT = 20
_MEM = 50
_EMB = 32
_BATCH = 1024

_NW = 32
_BATCH_W = _BATCH // _NW
_ROWS_CHUNK = _MEM * _SENT

_SCALE = 4.0 / (_EMB * _SENT)
_W = [float((s + 1) - (_SENT + 1) / 2.0) for s in range(_SENT - 1)]


def _sc_body(x_hbm, emb_hbm, temp_hbm, out_hbm,
             idx_v, rows_v, out_v, temp_v, gsem0, gsem1, osem0, osem1):
    wid = lax.axis_index("s") * 2 + lax.axis_index("c")
    b0 = wid * _BATCH_W

    pltpu.sync_copy(x_hbm.at[pl.ds(b0, _BATCH_W)], idx_v)
    pltpu.sync_copy(temp_hbm, temp_v)

    gsems = (gsem0, gsem1)
    osems = (osem0, osem1)

    lane = lax.iota(jnp.int32, 16).astype(jnp.float32)
    cvec = [(lane + float(h * 16) - (_EMB - 1) / 2.0) * _SCALE
            for h in range(2)]

    def fire(c, buf):
        for m in range(_MEM):
            pltpu.make_async_copy(
                emb_hbm.at[idx_v.at[c, m]],
                rows_v.at[buf, pl.ds(m * _SENT, _SENT)],
                gsems[buf],
            ).start()

    def drain(buf):
        pltpu.make_async_copy(
            emb_hbm.at[pl.ds(0, _ROWS_CHUNK)], rows_v.at[buf], gsems[buf]
        ).wait()

    def store(c, buf):
        return pltpu.make_async_copy(
            out_v.at[buf], out_hbm.at[b0 + c], osems[buf])

    def compute(buf):
        def seg_body(m, carry):
            rbase = m * _SENT
            for h in range(2):
                sl = pl.ds(h * 16, 16)
                acc = rows_v[buf, rbase, sl] * _W[0]
                for s in range(1, _SENT - 1):
                    acc = acc + rows_v[buf, rbase + s, sl] * _W[s]
                out_v[buf, m, sl] = (acc * cvec[h]
                                     + rows_v[buf, rbase + _SENT - 1, sl]
                                     + temp_v[m, sl])
            return carry
        lax.fori_loop(0, _MEM, seg_body, 0)

    fire(0, 0)
    fire(1, 1)
    for buf in range(2):
        drain(buf)
        compute(buf)
        store(buf, buf).start()
        fire(buf + 2, buf)

    def loop_body(i, carry):
        for buf in range(2):
            c = 2 * i + buf
            drain(buf)
            store(c - 2, buf).wait()
            compute(buf)
            store(c, buf).start()
            fire(c + 2, buf)
        return carry

    lax.fori_loop(1, _BATCH_W // 2 - 1, loop_body, 0)

    for buf in range(2):
        c = _BATCH_W - 2 + buf
        drain(buf)
        store(c - 2, buf).wait()
        compute(buf)
        store(c, buf).start()

    for buf in range(2):
        store(_BATCH_W - 2 + buf, buf).wait()


_sc_call = pl.kernel(
    _sc_body,
    out_type=jax.ShapeDtypeStruct((_BATCH, _MEM, _EMB), jnp.float32),
    mesh=plsc.VectorSubcoreMesh(core_axis_name="c", subcore_axis_name="s"),
    scratch_types=[
        pltpu.VMEM((_BATCH_W, _MEM, _SENT), jnp.int32),
        pltpu.VMEM((2, _ROWS_CHUNK, _EMB), jnp.float32),
        pltpu.VMEM((2, _MEM, _EMB), jnp.float32),
        pltpu.VMEM((_MEM, _EMB), jnp.float32),
        pltpu.SemaphoreType.DMA,
        pltpu.SemaphoreType.DMA,
        pltpu.SemaphoreType.DMA,
        pltpu.SemaphoreType.DMA,
    ],
    compiler_params=pltpu.CompilerParams(use_tc_tiling_on_sc=False),
)


@jax.jit
def kernel(x, emb_table, temporal_table):
    return _sc_call(x.astype(jnp.int32), emb_table, temporal_table)


# speedup vs baseline: 27.7450x; 1.3242x over previous
import jax
import jax.numpy as jnp
from jax import lax
from jax.experimental import pallas as pl
from jax.experimental.pallas import tpu as pltpu
from jax.experimental.pallas import tpu_sc as plsc

_VOCAB = 100000
_SENT = 20
_MEM = 50
_EMB = 32
_BATCH = 1024

_NW = 32
_BATCH_W = _BATCH // _NW
_ROWS_CHUNK = _SENT * _BATCH_W

_SCALE = 4.0 / (_EMB * _SENT)
_W = [float((s + 1) - (_SENT + 1) / 2.0) for s in range(_SENT - 1)]


def _sc_body(x_hbm, emb_hbm, temp_hbm, out_hbm,
             idx_v, rows_v, out_v, temp_v, gsem0, gsem1, osem0, osem1):
    wid = lax.axis_index("s") * 2 + lax.axis_index("c")
    b0 = wid * _BATCH_W

    pltpu.sync_copy(x_hbm.at[:, :, pl.ds(b0, _BATCH_W)], idx_v)
    pltpu.sync_copy(temp_hbm, temp_v)

    gsems = (gsem0, gsem1)
    osems = (osem0, osem1)

    lane = lax.iota(jnp.int32, 16)
    lane_f = lane.astype(jnp.float32)
    cvec = [(lane_f + float(h * 16) - (_EMB - 1) / 2.0) * _SCALE
            for h in range(2)]
    erow = [lane + h * 16 for h in range(2)]

    def fire(m, buf):
        for s in range(_SENT):
            pltpu.make_async_copy(
                emb_hbm.at[idx_v.at[s, m]],
                rows_v.at[buf, pl.ds(s * _BATCH_W, _BATCH_W)],
                gsems[buf],
            ).start()

    def drain(buf):
        pltpu.make_async_copy(
            emb_hbm.at[pl.ds(0, _ROWS_CHUNK)], rows_v.at[buf], gsems[buf]
        ).wait()

    def store(m, buf):
        return pltpu.make_async_copy(
            out_v.at[buf], out_hbm.at[m, :, pl.ds(b0, _BATCH_W)], osems[buf])

    def compute(m, buf):
        def b_body(b, carry):
            b_vec = jnp.full((16,), b, jnp.int32)
            for h in range(2):
                sl = pl.ds(h * 16, 16)
                acc = rows_v[buf, b, sl] * _W[0]
                for s in range(1, _SENT - 1):
                    acc = acc + rows_v[buf, s * _BATCH_W + b, sl] * _W[s]
                res = (acc * cvec[h]
                       + rows_v[buf, (_SENT - 1) * _BATCH_W + b, sl]
                       + temp_v[m, sl])
                plsc.store_scatter(out_v.at[buf], [erow[h], b_vec], res)
            return carry
        lax.fori_loop(0, _BATCH_W, b_body, 0)

    fire(0, 0)
    fire(1, 1)
    for buf in range(2):
        drain(buf)
        compute(buf, buf)
        store(buf, buf).start()
        fire(buf + 2, buf)

    def loop_body(i, carry):
        for buf in range(2):
            m = 2 * i + buf
            drain(buf)
            store(m - 2, buf).wait()
            compute(m, buf)
            store(m, buf).start()
            fire(m + 2, buf)
        return carry

    lax.fori_loop(1, _MEM // 2 - 1, loop_body, 0)

    for buf in range(2):
        m = _MEM - 2 + buf
        drain(buf)
        store(m - 2, buf).wait()
        compute(m, buf)
        store(m, buf).start()

    for buf in range(2):
        store(_MEM - 2 + buf, buf).wait()


_sc_call = pl.kernel(
    _sc_body,
    out_type=jax.ShapeDtypeStruct((_MEM, _EMB, _BATCH), jnp.float32),
    mesh=plsc.VectorSubcoreMesh(core_axis_name="c", subcore_axis_name="s"),
    scratch_types=[
        pltpu.VMEM((_SENT, _MEM, _BATCH_W), jnp.int32),
        pltpu.VMEM((2, _ROWS_CHUNK, _EMB), jnp.float32),
        pltpu.VMEM((2, _EMB, _BATCH_W), jnp.float32),
        pltpu.VMEM((_MEM, _EMB), jnp.float32),
        pltpu.SemaphoreType.DMA,
        pltpu.SemaphoreType.DMA,
        pltpu.SemaphoreType.DMA,
        pltpu.SemaphoreType.DMA,
    ],
    compiler_params=pltpu.CompilerParams(use_tc_tiling_on_sc=False,
                                         needs_layout_passes=False),
)


@jax.jit
def kernel(x, emb_table, temporal_table):
    xt = x.astype(jnp.int32).transpose(2, 1, 0)
    out_t = _sc_call(xt, emb_table, temporal_table)
    return out_t.transpose(2, 0, 1)
